# baseline (speedup 1.0000x reference)
import functools

import jax
import jax.numpy as jnp
from jax import lax
from jax.experimental import pallas as pl
from jax.experimental.pallas import tpu as pltpu
from jax.experimental.pallas import tpu_sc as plsc

_NUM_EMBEDDINGS = 100000
_EMBEDDING_DIM = 128
_TP_DEGREE = 4
_RANK = 1
_VOCAB_PER_RANK = _NUM_EMBEDDINGS // _TP_DEGREE
_VOCAB_START = _RANK * _VOCAB_PER_RANK
_VOCAB_END = (_RANK + 1) * _VOCAB_PER_RANK

_LANES = 16
_NW = 32
_CHUNK = 128


def _make_kernel(n_chunks):
    mesh = plsc.VectorSubcoreMesh(core_axis_name="c", subcore_axis_name="s")
    b_total = _NW * n_chunks * _CHUNK

    @functools.partial(
        pl.kernel,
        out_type=jax.ShapeDtypeStruct((b_total, _EMBEDDING_DIM), jnp.float32),
        mesh=mesh,
        scratch_types=[
            pltpu.VMEM((n_chunks, _CHUNK), jnp.int32),
            pltpu.VMEM((_CHUNK, _EMBEDDING_DIM), jnp.float32),
            pltpu.VMEM((_CHUNK, _EMBEDDING_DIM), jnp.float32),
            pltpu.SemaphoreType.DMA,
            pltpu.SemaphoreType.DMA,
            pltpu.SemaphoreType.DMA,
            pltpu.SemaphoreType.DMA,
        ],
    )
    def emb_kernel(ids_hbm, table_hbm, out_hbm, idx_v, buf0, buf1,
                   gsem0, gsem1, osem0, osem1):
        wid = lax.axis_index("s") * 2 + lax.axis_index("c")
        out_base = wid * (n_chunks * _CHUNK)

        pltpu.sync_copy(ids_hbm.at[wid], idx_v)

        vstart = jnp.full((_LANES,), _VOCAB_START, jnp.int32)
        nlocal = jnp.full((_LANES,), _VOCAB_PER_RANK, jnp.int32)

        def transform(j, carry):
            for i in range(_CHUNK // _LANES):
                v = idx_v[j, pl.ds(i * _LANES, _LANES)]
                local = v - vstart
                ok = (local >= 0) & (local < nlocal)
                idx_v[j, pl.ds(i * _LANES, _LANES)] = jnp.where(
                    ok, local, nlocal)
            return carry

        lax.fori_loop(0, n_chunks, transform, 0)

        bufs = (buf0, buf1)
        gsems = (gsem0, gsem1)
        osems = (osem0, osem1)

        def start_gather(j, slot):
            pltpu.async_copy(table_hbm.at[idx_v.at[j]], bufs[slot],
                             gsems[slot])

        def start_write(j, slot):
            pltpu.async_copy(
                bufs[slot],
                out_hbm.at[pl.ds(out_base + j * _CHUNK, _CHUNK)],
                osems[slot])

        def wait_gather(j, slot):
            pltpu.make_async_copy(table_hbm.at[idx_v.at[j]], bufs[slot],
                                  gsems[slot]).wait()

        def wait_write(j, slot):
            pltpu.make_async_copy(
                bufs[slot],
                out_hbm.at[pl.ds(out_base + j * _CHUNK, _CHUNK)],
                osems[slot]).wait()

        start_gather(0, 0)

        def pipe(t, carry):
            j0 = t * 2
            wait_gather(j0, 0)
            start_gather(j0 + 1, 1)
            start_write(j0, 0)
            wait_gather(j0 + 1, 1)

            @pl.when(t + 1 < n_chunks // 2)
            def _():
                start_gather(j0 + 2, 0)

            start_write(j0 + 1, 1)
            wait_write(j0, 0)
            wait_write(j0 + 1, 1)
            return carry

        lax.fori_loop(0, n_chunks // 2, pipe, 0)

    return emb_kernel


@jax.jit
def kernel(input_ids, weight):
    batch, seq = input_ids.shape
    b_total = batch * seq
    n_chunks = b_total // (_NW * _CHUNK)
    ids3 = input_ids.astype(jnp.int32).reshape(_NW, n_chunks, _CHUNK)
    table = jnp.zeros((_VOCAB_PER_RANK + 8, _EMBEDDING_DIM), jnp.float32)
    table = lax.dynamic_update_slice(table, weight, (0, 0))
    out = _make_kernel(n_chunks)(ids3, table)
    return out.reshape(batch, seq, _EMBEDDING_DIM)


# speedup vs baseline: 19.3167x; 19.3167x over previous
import functools

import jax
import jax.numpy as jnp
from jax import lax
from jax.experimental import pallas as pl
from jax.experimental.pallas import tpu as pltpu
from jax.experimental.pallas import tpu_sc as plsc

_NUM_EMBEDDINGS = 100000
_EMBEDDING_DIM = 128
_TP_DEGREE = 4
_RANK = 1
_VOCAB_PER_RANK = _NUM_EMBEDDINGS // _TP_DEGREE
_VOCAB_START = _RANK * _VOCAB_PER_RANK
_VOCAB_END = (_RANK + 1) * _VOCAB_PER_RANK

_LANES = 16
_NW = 32
_CHUNK = 128
_NBUF = 5
_ZPAD = 1024


def _make_kernel(n_chunks):
    mesh = plsc.VectorSubcoreMesh(core_axis_name="c", subcore_axis_name="s")
    b_total = _NW * n_chunks * _CHUNK
    n_outer = n_chunks // _NBUF
    assert n_outer * _NBUF == n_chunks

    @functools.partial(
        pl.kernel,
        out_type=jax.ShapeDtypeStruct((b_total, _EMBEDDING_DIM), jnp.float32),
        mesh=mesh,
        scratch_types=[
            pltpu.VMEM((n_chunks, _CHUNK), jnp.int32),
            *([pltpu.VMEM((_CHUNK, _EMBEDDING_DIM), jnp.float32)] * _NBUF),
            *([pltpu.SemaphoreType.DMA] * (2 * _NBUF)),
        ],
    )
    def emb_kernel(ids_hbm, table_hbm, out_hbm, idx_v, *bufs_and_sems):
        bufs = bufs_and_sems[:_NBUF]
        gsems = bufs_and_sems[_NBUF:2 * _NBUF]
        osems = bufs_and_sems[2 * _NBUF:]
        wid = lax.axis_index("s") * 2 + lax.axis_index("c")
        out_base = wid * (n_chunks * _CHUNK)

        pltpu.sync_copy(ids_hbm.at[wid], idx_v)

        vstart = jnp.full((_LANES,), _VOCAB_START, jnp.int32)
        nlocal = jnp.full((_LANES,), _VOCAB_PER_RANK, jnp.int32)
        zmask = jnp.full((_LANES,), _ZPAD - 1, jnp.int32)

        def transform(j, carry):
            for i in range(_CHUNK // _LANES):
                v = idx_v[j, pl.ds(i * _LANES, _LANES)]
                local = v - vstart
                ok = (local >= 0) & (local < nlocal)
                zrow = nlocal + (v & zmask)
                idx_v[j, pl.ds(i * _LANES, _LANES)] = jnp.where(
                    ok, local, zrow)
            return carry

        lax.fori_loop(0, n_chunks, transform, 0)

        def start_gather(j, slot):
            pltpu.async_copy(table_hbm.at[idx_v.at[j]], bufs[slot],
                             gsems[slot])

        def wait_gather(j, slot):
            pltpu.make_async_copy(table_hbm.at[idx_v.at[j]], bufs[slot],
                                  gsems[slot]).wait()

        def start_write(j, slot):
            pltpu.async_copy(
                bufs[slot],
                out_hbm.at[pl.ds(out_base + j * _CHUNK, _CHUNK)],
                osems[slot])

        def wait_write(j, slot):
            pltpu.make_async_copy(
                bufs[slot],
                out_hbm.at[pl.ds(out_base + j * _CHUNK, _CHUNK)],
                osems[slot]).wait()

        for b in range(_NBUF):
            start_gather(b, b)

        def pipe(t, carry):
            for b in range(_NBUF):
                j = t * _NBUF + b
                wait_gather(j, b)
                start_write(j, b)
                wait_write(j, b)

                @pl.when(t + 1 < n_outer)
                def _():
                    start_gather(j + _NBUF, b)
            return carry

        lax.fori_loop(0, n_outer, pipe, 0)

    return emb_kernel


@jax.jit
def kernel(input_ids, weight):
    batch, seq = input_ids.shape
    b_total = batch * seq
    n_chunks = b_total // (_NW * _CHUNK)
    ids3 = input_ids.astype(jnp.int32).reshape(_NW, n_chunks, _CHUNK)
    table = jnp.zeros((_VOCAB_PER_RANK + _ZPAD, _EMBEDDING_DIM), jnp.float32)
    table = lax.dynamic_update_slice(table, weight, (0, 0))
    out = _make_kernel(n_chunks)(ids3, table)
    return out.reshape(batch, seq, _EMBEDDING_DIM)


# speedup vs baseline: 21.0895x; 1.0918x over previous
import functools

import jax
import jax.numpy as jnp
from jax import lax
from jax.experimental import pallas as pl
from jax.experimental.pallas import tpu as pltpu
from jax.experimental.pallas import tpu_sc as plsc

_NUM_EMBEDDINGS = 100000
_EMBEDDING_DIM = 128
_TP_DEGREE = 4
_RANK = 1
_VOCAB_PER_RANK = _NUM_EMBEDDINGS // _TP_DEGREE
_VOCAB_START = _RANK * _VOCAB_PER_RANK
_VOCAB_END = (_RANK + 1) * _VOCAB_PER_RANK

_LANES = 16
_NW = 32
_CHUNK = 128
_NBUF = 5
_SPREAD = 16383


def _make_kernel(n_chunks):
    mesh = plsc.VectorSubcoreMesh(core_axis_name="c", subcore_axis_name="s")
    b_total = _NW * n_chunks * _CHUNK
    n_outer = n_chunks // _NBUF
    assert n_outer * _NBUF == n_chunks

    @functools.partial(
        pl.kernel,
        out_type=jax.ShapeDtypeStruct((b_total, _EMBEDDING_DIM), jnp.float32),
        mesh=mesh,
        scratch_types=[
            pltpu.VMEM((n_chunks, _CHUNK), jnp.int32),
            pltpu.VMEM((n_chunks, _CHUNK), jnp.int32),
            pltpu.VMEM((n_chunks, _CHUNK), jnp.int32),
            *([pltpu.VMEM((_CHUNK, _EMBEDDING_DIM), jnp.float32)] * _NBUF),
            *([pltpu.SemaphoreType.DMA] * (2 * _NBUF)),
        ],
    )
    def emb_kernel(ids_hbm, table_hbm, out_hbm, ids_v, gidx_v, bad_v,
                   *bufs_and_sems):
        bufs = bufs_and_sems[:_NBUF]
        gsems = bufs_and_sems[_NBUF:2 * _NBUF]
        osems = bufs_and_sems[2 * _NBUF:]
        wid = lax.axis_index("s") * 2 + lax.axis_index("c")
        out_base = wid * (n_chunks * _CHUNK)

        pltpu.sync_copy(ids_hbm.at[wid], ids_v)

        vstart = jnp.full((_LANES,), _VOCAB_START, jnp.int32)
        nlocal = jnp.full((_LANES,), _VOCAB_PER_RANK, jnp.int32)
        smask = jnp.full((_LANES,), _SPREAD, jnp.int32)

        def transform(j, carry):
            for i in range(_CHUNK // _LANES):
                v = ids_v[j, pl.ds(i * _LANES, _LANES)]
                local = v - vstart
                ok = (local >= 0) & (local < nlocal)
                gidx_v[j, pl.ds(i * _LANES, _LANES)] = jnp.where(
                    ok, local, v & smask)
                bad_v[j, pl.ds(i * _LANES, _LANES)] = jnp.where(
                    ok, jnp.zeros((_LANES,), jnp.int32),
                    jnp.full((_LANES,), 1, jnp.int32))
            return carry

        lax.fori_loop(0, n_chunks, transform, 0)

        def start_gather(j, slot):
            pltpu.async_copy(table_hbm.at[gidx_v.at[j]], bufs[slot],
                             gsems[slot])

        def wait_gather(j, slot):
            pltpu.make_async_copy(table_hbm.at[gidx_v.at[j]], bufs[slot],
                                  gsems[slot]).wait()

        def start_write(j, slot):
            pltpu.async_copy(
                bufs[slot],
                out_hbm.at[pl.ds(out_base + j * _CHUNK, _CHUNK)],
                osems[slot])

        def wait_write(j, slot):
            pltpu.make_async_copy(
                bufs[slot],
                out_hbm.at[pl.ds(out_base + j * _CHUNK, _CHUNK)],
                osems[slot]).wait()

        zero16 = jnp.zeros((_LANES,), jnp.float32)

        def zero_masked(j, buf):
            def grp_body(g, carry):
                bad16 = bad_v[j, pl.ds(g * _LANES, _LANES)]
                for l in range(_LANES):
                    row = g * _LANES + l

                    @pl.when(bad16[l] != 0)
                    def _():
                        for c in range(_EMBEDDING_DIM // _LANES):
                            buf[row, pl.ds(c * _LANES, _LANES)] = zero16
                return carry

            lax.fori_loop(0, _CHUNK // _LANES, grp_body, 0)

        for b in range(_NBUF):
            start_gather(b, b)

        def pipe(t, carry):
            for b in range(_NBUF):
                j = t * _NBUF + b
                wait_gather(j, b)
                zero_masked(j, bufs[b])
                start_write(j, b)
                wait_write(j, b)

                @pl.when(t + 1 < n_outer)
                def _():
                    start_gather(j + _NBUF, b)
            return carry

        lax.fori_loop(0, n_outer, pipe, 0)

    return emb_kernel


@jax.jit
def kernel(input_ids, weight):
    batch, seq = input_ids.shape
    b_total = batch * seq
    n_chunks = b_total // (_NW * _CHUNK)
    ids3 = input_ids.astype(jnp.int32).reshape(_NW, n_chunks, _CHUNK)
    out = _make_kernel(n_chunks)(ids3, weight)
    return out.reshape(batch, seq, _EMBEDDING_DIM)


# speedup vs baseline: 62.7190x; 2.9740x over previous
import functools

import jax
import jax.numpy as jnp
from jax import lax
from jax.experimental import pallas as pl
from jax.experimental.pallas import tpu as pltpu
from jax.experimental.pallas import tpu_sc as plsc

_NUM_EMBEDDINGS = 100000
_EMBEDDING_DIM = 128
_TP_DEGREE = 4
_RANK = 1
_VOCAB_PER_RANK = _NUM_EMBEDDINGS // _TP_DEGREE
_VOCAB_START = _RANK * _VOCAB_PER_RANK
_VOCAB_END = (_RANK + 1) * _VOCAB_PER_RANK

_LANES = 16
_NW = 32
_CHUNK = 128
_NBUF = 5
_SPREAD = 16383


def _make_kernel(n_chunks):
    mesh = plsc.VectorSubcoreMesh(core_axis_name="c", subcore_axis_name="s")
    b_total = _NW * n_chunks * _CHUNK
    n_outer = n_chunks // _NBUF
    assert n_outer * _NBUF == n_chunks

    @functools.partial(
        pl.kernel,
        out_type=jax.ShapeDtypeStruct((b_total, _EMBEDDING_DIM), jnp.float32),
        mesh=mesh,
        scratch_types=[
            pltpu.VMEM((n_chunks, _CHUNK), jnp.int32),
            pltpu.VMEM((n_chunks, _CHUNK), jnp.int32),
            pltpu.VMEM((n_chunks, _CHUNK), jnp.int32),
            *([pltpu.VMEM((_CHUNK, _EMBEDDING_DIM), jnp.float32)] * _NBUF),
            *([pltpu.SemaphoreType.DMA] * (2 * _NBUF)),
        ],
    )
    def emb_kernel(ids_hbm, table_hbm, out_hbm, ids_v, gidx_v, bad_v,
                   *bufs_and_sems):
        bufs = bufs_and_sems[:_NBUF]
        gsems = bufs_and_sems[_NBUF:2 * _NBUF]
        osems = bufs_and_sems[2 * _NBUF:]
        wid = lax.axis_index("s") * 2 + lax.axis_index("c")
        out_base = wid * (n_chunks * _CHUNK)

        pltpu.sync_copy(ids_hbm.at[wid], ids_v)

        vstart = jnp.full((_LANES,), _VOCAB_START, jnp.int32)
        nlocal = jnp.full((_LANES,), _VOCAB_PER_RANK, jnp.int32)
        smask = jnp.full((_LANES,), _SPREAD, jnp.int32)

        def transform(j, carry):
            for i in range(_CHUNK // _LANES):
                v = ids_v[j, pl.ds(i * _LANES, _LANES)]
                local = v - vstart
                ok = (local >= 0) & (local < nlocal)
                gidx_v[j, pl.ds(i * _LANES, _LANES)] = jnp.where(
                    ok, local, v & smask)
                bad_v[j, pl.ds(i * _LANES, _LANES)] = jnp.where(
                    ok, jnp.zeros((_LANES,), jnp.int32),
                    jnp.full((_LANES,), 1, jnp.int32))
            return carry

        lax.fori_loop(0, n_chunks, transform, 0)

        def start_gather(j, slot):
            pltpu.async_copy(table_hbm.at[gidx_v.at[j]], bufs[slot],
                             gsems[slot])

        def wait_gather(j, slot):
            pltpu.make_async_copy(table_hbm.at[gidx_v.at[j]], bufs[slot],
                                  gsems[slot]).wait()

        def start_write(j, slot):
            pltpu.async_copy(
                bufs[slot],
                out_hbm.at[pl.ds(out_base + j * _CHUNK, _CHUNK)],
                osems[slot])

        def wait_write(j, slot):
            pltpu.make_async_copy(
                bufs[slot],
                out_hbm.at[pl.ds(out_base + j * _CHUNK, _CHUNK)],
                osems[slot]).wait()

        zero16 = jnp.zeros((_LANES,), jnp.float32)

        def zero_masked(j, buf):
            def grp_body(g, carry):
                bad16 = bad_v[j, pl.ds(g * _LANES, _LANES)]
                for l in range(_LANES):
                    row = g * _LANES + l

                    @pl.when(bad16[l] != 0)
                    def _():
                        for c in range(_EMBEDDING_DIM // _LANES):
                            buf[row, pl.ds(c * _LANES, _LANES)] = zero16
                return carry

            lax.fori_loop(0, _CHUNK // _LANES, grp_body, 0)

        for b in range(_NBUF):
            start_gather(b, b)

        def pipe(t, carry):
            for b in range(_NBUF):
                j = t * _NBUF + b
                wait_gather(j, b)
                zero_masked(j, bufs[b])
                start_write(j, b)
                wait_write(j, b)

                @pl.when(t + 1 < n_outer)
                def _():
                    start_gather(j + _NBUF, b)
            return carry

        lax.fori_loop(0, n_outer, pipe, 0)

    return emb_kernel


@jax.jit
def kernel(input_ids, weight):
    batch, seq = input_ids.shape
    b_total = batch * seq
    n_chunks = b_total // (_NW * _CHUNK)
    ids3 = input_ids.astype(jnp.int32).T.reshape(_NW, n_chunks, _CHUNK)
    out = _make_kernel(n_chunks)(ids3, weight)
    return out.reshape(seq, batch, _EMBEDDING_DIM).transpose(1, 0, 2)
